# baseline (speedup 1.0000x reference)
import functools

import jax
import jax.numpy as jnp
from jax import lax
from jax.experimental import pallas as pl
from jax.experimental.pallas import tpu as pltpu
from jax.experimental.pallas import tpu_sc as plsc

NUM_WORKERS = 32
CHUNK = 2048
LANES = 16
NBUF = 2


def _make_sc_call(n, nx, ny, nz):
    n_chunks = -(-n // CHUNK)
    total_chunks = -(-n_chunks // (2 * NUM_WORKERS)) * (2 * NUM_WORKERS)
    cpw = total_chunks // NUM_WORKERS
    last_base = n - CHUNK
    sx = ny * nz
    sy = nz
    nc = nx * ny * nz

    mesh = plsc.VectorSubcoreMesh(core_axis_name="c", subcore_axis_name="s")

    bank_scratch = (
        [pltpu.VMEM((CHUNK,), jnp.float32) for _ in range(3)]
        + [pltpu.VMEM((CHUNK,), jnp.float32) for _ in range(3)]
        + [pltpu.VMEM((CHUNK,), jnp.int32) for _ in range(4)]
        + [pltpu.VMEM((1, CHUNK), jnp.int32) for _ in range(8)]
        + [pltpu.VMEM((1, CHUNK), jnp.int32) for _ in range(8)]
        + [pltpu.SemaphoreType.DMA]
    )
    scratch = (
        bank_scratch * NBUF
        + [pltpu.VMEM((CHUNK,), jnp.float32) for _ in range(2)]
        + [pltpu.VMEM((LANES,), jnp.float32) for _ in range(6)]
    )

    @functools.partial(
        pl.kernel,
        mesh=mesh,
        out_type=(
            jax.ShapeDtypeStruct((n,), jnp.float32),
            jax.ShapeDtypeStruct((n,), jnp.float32),
        ),
        scratch_types=scratch,
    )
    def sc_call(posx_h, posy_h, posz_h, par_h, tab_h,
                outm_h, outs_h, *refs):
        p0_h = tab_h.at[pl.ds(0, nc)]
        p1_h = tab_h.at[pl.ds(nc, nc)]
        nb = 27
        banks = []
        for b in range(NBUF):
            r = refs[b * nb:(b + 1) * nb]
            banks.append(dict(pos=r[0:3], frac=r[3:6], idx=r[6:10],
                              res=r[10:18], lo=r[18:26], sem=r[26]))
        out_v = refs[2 * nb:2 * nb + 2]
        par_v = refs[2 * nb + 2:2 * nb + 8]

        wid = lax.axis_index("s") * 2 + lax.axis_index("c")

        for d in range(6):
            pltpu.sync_copy(par_h.at[pl.ds(d * LANES, LANES)], par_v[d])
        minx = par_v[0][:]
        miny = par_v[1][:]
        minz = par_v[2][:]
        sclx = par_v[3][:]
        scly = par_v[4][:]
        sclz = par_v[5][:]

        def chunk_base(j):
            t = j * NUM_WORKERS + wid
            return jnp.minimum(t * CHUNK, last_base)

        def load_and_index(j, bk):
            base = chunk_base(j)
            pltpu.sync_copy(posx_h.at[pl.ds(base, CHUNK)], bk["pos"][0])
            pltpu.sync_copy(posy_h.at[pl.ds(base, CHUNK)], bk["pos"][1])
            pltpu.sync_copy(posz_h.at[pl.ds(base, CHUNK)], bk["pos"][2])

            def index_body(g, c):
                sl = pl.ds(g * LANES, LANES)
                fx = jnp.maximum((bk["pos"][0][sl] - minx) * sclx, 0.0)
                fy = jnp.maximum((bk["pos"][1][sl] - miny) * scly, 0.0)
                fz = jnp.maximum((bk["pos"][2][sl] - minz) * sclz, 0.0)
                x0 = jnp.minimum(fx.astype(jnp.int32), nx - 2)
                y0 = jnp.minimum(fy.astype(jnp.int32), ny - 2)
                z0 = jnp.minimum(fz.astype(jnp.int32), nz - 2)
                bk["frac"][0][sl] = fx - x0.astype(jnp.float32)
                bk["frac"][1][sl] = fy - y0.astype(jnp.float32)
                bk["frac"][2][sl] = fz - z0.astype(jnp.float32)
                b = x0 * sx + y0 * sy + z0
                bk["idx"][0][sl] = b
                bk["idx"][1][sl] = b + sy
                bk["idx"][2][sl] = b + sx
                bk["idx"][3][sl] = b + (sx + sy)
                return c

            lax.fori_loop(0, CHUNK // LANES, index_body, 0)

        def gathers(bk):
            return (
                [pltpu.make_async_copy(p0_h.at[bk["idx"][k]],
                                       bk["res"][k].at[0], bk["sem"])
                 for k in range(4)]
                + [pltpu.make_async_copy(p1_h.at[bk["idx"][k]],
                                         bk["res"][4 + k].at[0], bk["sem"])
                   for k in range(4)]
            )

        def fire(bk):
            for cp in gathers(bk):
                cp.start()

        def drain(bk):
            for cp in gathers(bk):
                cp.wait()

        def combine_store(j, bk):
            base = chunk_base(j)
            res_v = bk["res"]
            lo_v = bk["lo"]
            res_f = [r.bitcast(jnp.float32) for r in res_v]
            lo_f = [r.bitcast(jnp.float32) for r in lo_v]

            def split_body(g, c):
                sl = pl.ds(g * LANES, LANES)
                for k in range(8):
                    w = res_v[k][0, sl]
                    lo_v[k][0, sl] = w << 16
                    res_v[k][0, sl] = w & jnp.int32(-65536)
                return c

            lax.fori_loop(0, CHUNK // LANES, split_body, 0)

            def combine_body(g, c):
                sl = pl.ds(g * LANES, LANES)
                tx = bk["frac"][0][sl]
                ty = bk["frac"][1][sl]
                tz = bk["frac"][2][sl]
                ux = 1.0 - tx
                uy = 1.0 - ty
                uz = 1.0 - tz
                q0 = ux * uy
                q1 = ux * ty
                q2 = tx * uy
                q3 = tx * ty

                def edge(k):
                    return lo_f[k][0, sl] * uz + res_f[k][0, sl] * tz

                m = (q0 * edge(0) + q1 * edge(1)
                     + q2 * edge(2) + q3 * edge(3))
                s = (q0 * edge(4) + q1 * edge(5)
                     + q2 * edge(6) + q3 * edge(7))
                out_v[0][sl] = m
                out_v[1][sl] = s
                return c

            lax.fori_loop(0, CHUNK // LANES, combine_body, 0)
            pltpu.sync_copy(out_v[0], outm_h.at[pl.ds(base, CHUNK)])
            pltpu.sync_copy(out_v[1], outs_h.at[pl.ds(base, CHUNK)])

        load_and_index(0, banks[0])
        fire(banks[0])

        def pair_body(p, carry):
            j0 = 2 * p
            j1 = j0 + 1
            j2 = j0 + 2
            load_and_index(j1, banks[1])
            fire(banks[1])
            drain(banks[0])
            combine_store(j0, banks[0])

            @pl.when(j2 < cpw)
            def _():
                load_and_index(j2, banks[0])
                fire(banks[0])

            drain(banks[1])
            combine_store(j1, banks[1])
            return carry

        lax.fori_loop(0, cpw // 2, pair_body, 0)

    return sc_call


def kernel(pos, grid, min_bound, max_bound):
    n = pos.shape[0]
    _, nx, ny, nz = grid.shape

    n_al = -(-n // 8) * 8
    if n_al != n:
        pos = jnp.concatenate([pos, pos[: n_al - n]])

    posx = pos[:, 0]
    posy = pos[:, 1]
    posz = pos[:, 2]

    grid_range = jnp.clip(max_bound - min_bound, 1e-6, None)
    dims = jnp.array([nx - 1, ny - 1, nz - 1], dtype=jnp.float32)
    scales = dims / grid_range
    params = jnp.concatenate(
        [
            jnp.repeat(min_bound.astype(jnp.float32), LANES),
            jnp.repeat(scales.astype(jnp.float32), LANES),
        ]
    )

    gb = grid.astype(jnp.bfloat16)
    u = lax.bitcast_convert_type(gb, jnp.uint16).astype(jnp.uint32)
    hi = jnp.concatenate([u[..., 1:], u[..., -1:]], axis=-1)
    table = lax.bitcast_convert_type(u | (hi << 16), jnp.int32).reshape(-1)

    sc_call = _make_sc_call(n_al, nx, ny, nz)
    outm, outs = sc_call(posx, posy, posz, params, table)
    if n_al != n:
        return (outm[:n], outs[:n])
    return (outm, outs)


# speedup vs baseline: 1.2268x; 1.2268x over previous
import functools

import jax
import jax.numpy as jnp
from jax import lax
from jax.experimental import pallas as pl
from jax.experimental.pallas import tpu as pltpu
from jax.experimental.pallas import tpu_sc as plsc

NUM_WORKERS = 32
CHUNK = 2048
LANES = 16
NBUF = 2


def _make_sc_call(n, nx, ny, nz):
    n_chunks = -(-n // CHUNK)
    total_chunks = -(-n_chunks // (2 * NUM_WORKERS)) * (2 * NUM_WORKERS)
    cpw = total_chunks // NUM_WORKERS
    last_base = n - CHUNK
    sx = ny * nz
    sy = nz
    nc = nx * ny * nz

    mesh = plsc.VectorSubcoreMesh(core_axis_name="c", subcore_axis_name="s")

    bank_scratch = (
        [pltpu.VMEM((CHUNK,), jnp.float32) for _ in range(3)]
        + [pltpu.VMEM((CHUNK,), jnp.float32) for _ in range(3)]
        + [pltpu.VMEM((CHUNK,), jnp.int32) for _ in range(4)]
        + [pltpu.VMEM((1, CHUNK), jnp.int32) for _ in range(8)]
        + [pltpu.VMEM((1, CHUNK), jnp.int32) for _ in range(8)]
        + [pltpu.SemaphoreType.DMA]
    )
    scratch = (
        bank_scratch * NBUF
        + [pltpu.VMEM((CHUNK,), jnp.float32) for _ in range(2)]
        + [pltpu.VMEM((LANES,), jnp.float32) for _ in range(6)]
    )

    @functools.partial(
        pl.kernel,
        mesh=mesh,
        out_type=(
            jax.ShapeDtypeStruct((n,), jnp.float32),
            jax.ShapeDtypeStruct((n,), jnp.float32),
        ),
        scratch_types=scratch,
    )
    def sc_call(posx_h, posy_h, posz_h, par_h, tab_h,
                outm_h, outs_h, *refs):
        p0_h = tab_h.at[pl.ds(0, nc)]
        p1_h = tab_h.at[pl.ds(nc, nc)]
        nb = 27
        banks = []
        for b in range(NBUF):
            r = refs[b * nb:(b + 1) * nb]
            banks.append(dict(pos=r[0:3], frac=r[3:6], idx=r[6:10],
                              res=r[10:18], lo=r[18:26], sem=r[26]))
        out_v = refs[2 * nb:2 * nb + 2]
        par_v = refs[2 * nb + 2:2 * nb + 8]

        wid = lax.axis_index("s") * 2 + lax.axis_index("c")

        for d in range(6):
            pltpu.sync_copy(par_h.at[pl.ds(d * LANES, LANES)], par_v[d])
        minx = par_v[0][:]
        miny = par_v[1][:]
        minz = par_v[2][:]
        sclx = par_v[3][:]
        scly = par_v[4][:]
        sclz = par_v[5][:]

        def chunk_base(j):
            t = j * NUM_WORKERS + wid
            return jnp.minimum(t * CHUNK, last_base)

        def load_and_index(j, bk):
            base = chunk_base(j)
            pltpu.sync_copy(posx_h.at[pl.ds(base, CHUNK)], bk["pos"][0])
            pltpu.sync_copy(posy_h.at[pl.ds(base, CHUNK)], bk["pos"][1])
            pltpu.sync_copy(posz_h.at[pl.ds(base, CHUNK)], bk["pos"][2])

            def index_body(g, c):
                sl = pl.ds(g * LANES, LANES)
                fx = jnp.maximum((bk["pos"][0][sl] - minx) * sclx, 0.0)
                fy = jnp.maximum((bk["pos"][1][sl] - miny) * scly, 0.0)
                fz = jnp.maximum((bk["pos"][2][sl] - minz) * sclz, 0.0)
                x0 = jnp.minimum(fx.astype(jnp.int32), nx - 2)
                y0 = jnp.minimum(fy.astype(jnp.int32), ny - 2)
                z0 = jnp.minimum(fz.astype(jnp.int32), nz - 2)
                bk["frac"][0][sl] = fx - x0.astype(jnp.float32)
                bk["frac"][1][sl] = fy - y0.astype(jnp.float32)
                bk["frac"][2][sl] = fz - z0.astype(jnp.float32)
                b = x0 * sx + y0 * sy + z0
                bk["idx"][0][sl] = b
                bk["idx"][1][sl] = b + sy
                bk["idx"][2][sl] = b + sx
                bk["idx"][3][sl] = b + (sx + sy)
                return c

            lax.fori_loop(0, CHUNK // LANES, index_body, 0)

        def gathers(bk):
            return (
                [pltpu.make_async_copy(p0_h.at[bk["idx"][k]],
                                       bk["res"][k].at[0], bk["sem"])
                 for k in range(4)]
                + [pltpu.make_async_copy(p1_h.at[bk["idx"][k]],
                                         bk["res"][4 + k].at[0], bk["sem"])
                   for k in range(4)]
            )

        def fire(bk):
            for cp in gathers(bk):
                cp.start()

        def drain(bk):
            for cp in gathers(bk):
                cp.wait()

        def combine_store(j, bk):
            base = chunk_base(j)
            res_v = bk["res"]
            lo_v = bk["lo"]
            res_f = [r.bitcast(jnp.float32) for r in res_v]
            lo_f = [r.bitcast(jnp.float32) for r in lo_v]

            def split_body(g, c):
                sl = pl.ds(g * LANES, LANES)
                for k in range(8):
                    w = res_v[k][0, sl]
                    lo_v[k][0, sl] = w << 16
                    res_v[k][0, sl] = w & jnp.int32(-65536)
                return c

            lax.fori_loop(0, CHUNK // LANES, split_body, 0)

            def combine_body(g, c):
                sl = pl.ds(g * LANES, LANES)
                tx = bk["frac"][0][sl]
                ty = bk["frac"][1][sl]
                tz = bk["frac"][2][sl]
                ux = 1.0 - tx
                uy = 1.0 - ty
                uz = 1.0 - tz
                q0 = ux * uy
                q1 = ux * ty
                q2 = tx * uy
                q3 = tx * ty

                def edge(k):
                    return lo_f[k][0, sl] * uz + res_f[k][0, sl] * tz

                m = (q0 * edge(0) + q1 * edge(1)
                     + q2 * edge(2) + q3 * edge(3))
                s = (q0 * edge(4) + q1 * edge(5)
                     + q2 * edge(6) + q3 * edge(7))
                out_v[0][sl] = m
                out_v[1][sl] = s
                return c

            lax.fori_loop(0, CHUNK // LANES, combine_body, 0)
            pltpu.sync_copy(out_v[0], outm_h.at[pl.ds(base, CHUNK)])
            pltpu.sync_copy(out_v[1], outs_h.at[pl.ds(base, CHUNK)])

        load_and_index(0, banks[0])
        fire(banks[0])

        def pair_body(p, carry):
            j0 = 2 * p
            j1 = j0 + 1
            j2 = j0 + 2
            load_and_index(j1, banks[1])
            fire(banks[1])
            drain(banks[0])
            combine_store(j0, banks[0])

            @pl.when(j2 < cpw)
            def _():
                load_and_index(j2, banks[0])
                fire(banks[0])

            drain(banks[1])
            combine_store(j1, banks[1])
            return carry

        lax.fori_loop(0, cpw // 2, pair_body, 0)

    return sc_call


def _pack_pairs_tc(grid, nx, ny, nz):
    rows = 2 * nx * ny
    blk = 512

    def body(g_ref, o_ref):
        x = g_ref[...]
        lo = lax.bitcast_convert_type(
            x.astype(jnp.bfloat16), jnp.uint16).astype(jnp.uint32)
        hi = pltpu.roll(lo, nz - 1, 1)
        o_ref[...] = lax.bitcast_convert_type(lo | (hi << 16), jnp.int32)

    g2 = grid.reshape(rows, nz)
    out = pl.pallas_call(
        body,
        grid=(rows // blk,),
        in_specs=[pl.BlockSpec((blk, nz), lambda i: (i, 0))],
        out_specs=pl.BlockSpec((blk, nz), lambda i: (i, 0)),
        out_shape=jax.ShapeDtypeStruct((rows, nz), jnp.int32),
    )(g2)
    return out.reshape(-1)


def kernel(pos, grid, min_bound, max_bound):
    n = pos.shape[0]
    _, nx, ny, nz = grid.shape

    n_al = -(-n // 8) * 8
    if n_al != n:
        pos = jnp.concatenate([pos, pos[: n_al - n]])

    posx = pos[:, 0]
    posy = pos[:, 1]
    posz = pos[:, 2]

    grid_range = jnp.clip(max_bound - min_bound, 1e-6, None)
    dims = jnp.array([nx - 1, ny - 1, nz - 1], dtype=jnp.float32)
    scales = dims / grid_range
    params = jnp.concatenate(
        [
            jnp.repeat(min_bound.astype(jnp.float32), LANES),
            jnp.repeat(scales.astype(jnp.float32), LANES),
        ]
    )

    table = _pack_pairs_tc(grid, nx, ny, nz)

    sc_call = _make_sc_call(n_al, nx, ny, nz)
    outm, outs = sc_call(posx, posy, posz, params, table)
    if n_al != n:
        return (outm[:n], outs[:n])
    return (outm, outs)


# speedup vs baseline: 1.4429x; 1.1761x over previous
import functools

import jax
import jax.numpy as jnp
from jax import lax
from jax.experimental import pallas as pl
from jax.experimental.pallas import tpu as pltpu
from jax.experimental.pallas import tpu_sc as plsc

NUM_WORKERS = 32
CHUNK = 2048
LANES = 16
NBUF = 2


def _make_sc_call(n, nx, ny, nz):
    n_chunks = -(-n // CHUNK)
    total_chunks = -(-n_chunks // (2 * NUM_WORKERS)) * (2 * NUM_WORKERS)
    cpw = total_chunks // NUM_WORKERS
    last_base = n - CHUNK
    sx = ny * nz
    sy = nz
    nc = nx * ny * nz

    mesh = plsc.VectorSubcoreMesh(core_axis_name="c", subcore_axis_name="s")

    bank_scratch = (
        [pltpu.VMEM((CHUNK,), jnp.float32) for _ in range(3)]
        + [pltpu.VMEM((CHUNK,), jnp.float32) for _ in range(3)]
        + [pltpu.VMEM((CHUNK,), jnp.int32) for _ in range(4)]
        + [pltpu.VMEM((1, CHUNK), jnp.int32) for _ in range(8)]
        + [pltpu.VMEM((1, CHUNK), jnp.int32) for _ in range(8)]
        + [pltpu.SemaphoreType.DMA]
    )
    scratch = (
        bank_scratch * NBUF
        + [pltpu.VMEM((CHUNK,), jnp.float32) for _ in range(2)]
        + [pltpu.VMEM((LANES,), jnp.float32) for _ in range(6)]
    )

    @functools.partial(
        pl.kernel,
        mesh=mesh,
        out_type=(
            jax.ShapeDtypeStruct((n,), jnp.float32),
            jax.ShapeDtypeStruct((n,), jnp.float32),
        ),
        scratch_types=scratch,
    )
    def sc_call(posx_h, posy_h, posz_h, par_h, tab_h,
                outm_h, outs_h, *refs):
        p0_h = tab_h.at[pl.ds(0, nc)]
        p1_h = tab_h.at[pl.ds(nc, nc)]
        nb = 27
        banks = []
        for b in range(NBUF):
            r = refs[b * nb:(b + 1) * nb]
            banks.append(dict(pos=r[0:3], frac=r[3:6], idx=r[6:10],
                              res=r[10:18], lo=r[18:26], sem=r[26]))
        out_v = refs[2 * nb:2 * nb + 2]
        par_v = refs[2 * nb + 2:2 * nb + 8]

        wid = lax.axis_index("s") * 2 + lax.axis_index("c")

        for d in range(6):
            pltpu.sync_copy(par_h.at[pl.ds(d * LANES, LANES)], par_v[d])
        minx = par_v[0][:]
        miny = par_v[1][:]
        minz = par_v[2][:]
        sclx = par_v[3][:]
        scly = par_v[4][:]
        sclz = par_v[5][:]

        def chunk_base(j):
            t = j * NUM_WORKERS + wid
            return jnp.minimum(t * CHUNK, last_base)

        def load_and_index(j, bk):
            base = chunk_base(j)
            pltpu.sync_copy(posx_h.at[pl.ds(base, CHUNK)], bk["pos"][0])
            pltpu.sync_copy(posy_h.at[pl.ds(base, CHUNK)], bk["pos"][1])
            pltpu.sync_copy(posz_h.at[pl.ds(base, CHUNK)], bk["pos"][2])

            def index_body(g, c):
                sl = pl.ds(g * LANES, LANES)
                fx = jnp.maximum((bk["pos"][0][sl] - minx) * sclx, 0.0)
                fy = jnp.maximum((bk["pos"][1][sl] - miny) * scly, 0.0)
                fz = jnp.maximum((bk["pos"][2][sl] - minz) * sclz, 0.0)
                x0 = jnp.minimum(fx.astype(jnp.int32), nx - 2)
                y0 = jnp.minimum(fy.astype(jnp.int32), ny - 2)
                z0 = jnp.minimum(fz.astype(jnp.int32), nz - 2)
                bk["frac"][0][sl] = fx - x0.astype(jnp.float32)
                bk["frac"][1][sl] = fy - y0.astype(jnp.float32)
                bk["frac"][2][sl] = fz - z0.astype(jnp.float32)
                b = x0 * sx + y0 * sy + z0
                bk["idx"][0][sl] = b
                bk["idx"][1][sl] = b + sy
                bk["idx"][2][sl] = b + sx
                bk["idx"][3][sl] = b + (sx + sy)
                return c

            lax.fori_loop(0, CHUNK // LANES, index_body, 0)

        def gathers(bk):
            return (
                [pltpu.make_async_copy(p0_h.at[bk["idx"][k]],
                                       bk["res"][k].at[0], bk["sem"])
                 for k in range(4)]
                + [pltpu.make_async_copy(p1_h.at[bk["idx"][k]],
                                         bk["res"][4 + k].at[0], bk["sem"])
                   for k in range(4)]
            )

        def fire(bk):
            for cp in gathers(bk):
                cp.start()

        def drain(bk):
            for cp in gathers(bk):
                cp.wait()

        def combine_store(j, bk):
            base = chunk_base(j)
            res_v = bk["res"]
            lo_v = bk["lo"]
            res_f = [r.bitcast(jnp.float32) for r in res_v]
            lo_f = [r.bitcast(jnp.float32) for r in lo_v]

            def split_body(g, c):
                sl = pl.ds(g * LANES, LANES)
                for k in range(8):
                    w = res_v[k][0, sl]
                    lo_v[k][0, sl] = w << 16
                    res_v[k][0, sl] = w & jnp.int32(-65536)
                return c

            lax.fori_loop(0, CHUNK // LANES, split_body, 0)

            def combine_body(g, c):
                sl = pl.ds(g * LANES, LANES)
                tx = bk["frac"][0][sl]
                ty = bk["frac"][1][sl]
                tz = bk["frac"][2][sl]
                ux = 1.0 - tx
                uy = 1.0 - ty
                uz = 1.0 - tz
                q0 = ux * uy
                q1 = ux * ty
                q2 = tx * uy
                q3 = tx * ty

                def edge(k):
                    return lo_f[k][0, sl] * uz + res_f[k][0, sl] * tz

                m = (q0 * edge(0) + q1 * edge(1)
                     + q2 * edge(2) + q3 * edge(3))
                s = (q0 * edge(4) + q1 * edge(5)
                     + q2 * edge(6) + q3 * edge(7))
                out_v[0][sl] = m
                out_v[1][sl] = s
                return c

            lax.fori_loop(0, CHUNK // LANES, combine_body, 0)
            pltpu.sync_copy(out_v[0], outm_h.at[pl.ds(base, CHUNK)])
            pltpu.sync_copy(out_v[1], outs_h.at[pl.ds(base, CHUNK)])

        load_and_index(0, banks[0])
        fire(banks[0])

        def pair_body(p, carry):
            j0 = 2 * p
            j1 = j0 + 1
            j2 = j0 + 2
            load_and_index(j1, banks[1])
            fire(banks[1])
            drain(banks[0])
            combine_store(j0, banks[0])

            @pl.when(j2 < cpw)
            def _():
                load_and_index(j2, banks[0])
                fire(banks[0])

            drain(banks[1])
            combine_store(j1, banks[1])
            return carry

        lax.fori_loop(0, cpw // 2, pair_body, 0)

    return sc_call


def _pack_pairs_tc(grid, nx, ny, nz):
    rows = 2 * nx * ny
    blk = 2048

    def body(g_ref, o_ref):
        x = g_ref[...]
        lo = lax.bitcast_convert_type(
            x.astype(jnp.bfloat16), jnp.uint16).astype(jnp.uint32)
        hi = pltpu.roll(lo, nz - 1, 1)
        o_ref[...] = lax.bitcast_convert_type(lo | (hi << 16), jnp.int32)

    g2 = grid.reshape(rows, nz)
    out = pl.pallas_call(
        body,
        grid=(rows // blk,),
        in_specs=[pl.BlockSpec((blk, nz), lambda i: (i, 0))],
        out_specs=pl.BlockSpec((blk, nz), lambda i: (i, 0)),
        out_shape=jax.ShapeDtypeStruct((rows, nz), jnp.int32),
    )(g2)
    return out.reshape(-1)


def kernel(pos, grid, min_bound, max_bound):
    n = pos.shape[0]
    _, nx, ny, nz = grid.shape

    n_al = -(-n // 8) * 8
    if n_al != n:
        pos = jnp.concatenate([pos, pos[: n_al - n]])

    posx = pos[:, 0]
    posy = pos[:, 1]
    posz = pos[:, 2]

    grid_range = jnp.clip(max_bound - min_bound, 1e-6, None)
    dims = jnp.array([nx - 1, ny - 1, nz - 1], dtype=jnp.float32)
    scales = dims / grid_range
    params = jnp.concatenate(
        [
            jnp.repeat(min_bound.astype(jnp.float32), LANES),
            jnp.repeat(scales.astype(jnp.float32), LANES),
        ]
    )

    table = _pack_pairs_tc(grid, nx, ny, nz)

    sc_call = _make_sc_call(n_al, nx, ny, nz)
    outm, outs = sc_call(posx, posy, posz, params, table)
    if n_al != n:
        return (outm[:n], outs[:n])
    return (outm, outs)


# speedup vs baseline: 1.4841x; 1.0285x over previous
import functools

import jax
import jax.numpy as jnp
from jax import lax
from jax.experimental import pallas as pl
from jax.experimental.pallas import tpu as pltpu
from jax.experimental.pallas import tpu_sc as plsc

NUM_WORKERS = 32
CHUNK = 2048
LANES = 16
NBUF = 2


def _make_sc_call(n, nx, ny, nz):
    n_chunks = -(-n // CHUNK)
    total_chunks = -(-n_chunks // (2 * NUM_WORKERS)) * (2 * NUM_WORKERS)
    cpw = total_chunks // NUM_WORKERS
    last_base = n - CHUNK
    sx = ny * nz
    sy = nz
    nc = nx * ny * nz

    mesh = plsc.VectorSubcoreMesh(core_axis_name="c", subcore_axis_name="s")

    bank_scratch = (
        [pltpu.VMEM((CHUNK,), jnp.float32) for _ in range(3)]
        + [pltpu.VMEM((CHUNK,), jnp.float32) for _ in range(3)]
        + [pltpu.VMEM((CHUNK,), jnp.int32) for _ in range(4)]
        + [pltpu.VMEM((1, CHUNK), jnp.int32) for _ in range(8)]
        + [pltpu.VMEM((1, CHUNK), jnp.int32) for _ in range(8)]
        + [pltpu.SemaphoreType.DMA]
    )
    scratch = (
        bank_scratch * NBUF
        + [pltpu.VMEM((CHUNK,), jnp.float32) for _ in range(2)]
        + [pltpu.VMEM((LANES,), jnp.float32) for _ in range(6)]
    )

    @functools.partial(
        pl.kernel,
        mesh=mesh,
        out_type=(
            jax.ShapeDtypeStruct((n,), jnp.float32),
            jax.ShapeDtypeStruct((n,), jnp.float32),
        ),
        scratch_types=scratch,
    )
    def sc_call(posx_h, posy_h, posz_h, par_h, tab_h,
                outm_h, outs_h, *refs):
        p0_h = tab_h.at[pl.ds(0, nc)]
        p1_h = tab_h.at[pl.ds(nc, nc)]
        nb = 27
        banks = []
        for b in range(NBUF):
            r = refs[b * nb:(b + 1) * nb]
            banks.append(dict(pos=r[0:3], frac=r[3:6], idx=r[6:10],
                              res=r[10:18], lo=r[18:26], sem=r[26]))
        out_v = refs[2 * nb:2 * nb + 2]
        par_v = refs[2 * nb + 2:2 * nb + 8]

        wid = lax.axis_index("s") * 2 + lax.axis_index("c")

        for d in range(6):
            pltpu.sync_copy(par_h.at[pl.ds(d * LANES, LANES)], par_v[d])
        minx = par_v[0][:]
        miny = par_v[1][:]
        minz = par_v[2][:]
        sclx = par_v[3][:]
        scly = par_v[4][:]
        sclz = par_v[5][:]

        def chunk_base(j):
            t = j * NUM_WORKERS + wid
            return jnp.minimum(t * CHUNK, last_base)

        def load_and_index(j, bk):
            base = chunk_base(j)
            pltpu.sync_copy(posx_h.at[pl.ds(base, CHUNK)], bk["pos"][0])
            pltpu.sync_copy(posy_h.at[pl.ds(base, CHUNK)], bk["pos"][1])
            pltpu.sync_copy(posz_h.at[pl.ds(base, CHUNK)], bk["pos"][2])

            def index_body(g, c):
                sl = pl.ds(g * LANES, LANES)
                fx = jnp.maximum((bk["pos"][0][sl] - minx) * sclx, 0.0)
                fy = jnp.maximum((bk["pos"][1][sl] - miny) * scly, 0.0)
                fz = jnp.maximum((bk["pos"][2][sl] - minz) * sclz, 0.0)
                x0 = jnp.minimum(fx.astype(jnp.int32), nx - 2)
                y0 = jnp.minimum(fy.astype(jnp.int32), ny - 2)
                z0 = jnp.minimum(fz.astype(jnp.int32), nz - 2)
                bk["frac"][0][sl] = fx - x0.astype(jnp.float32)
                bk["frac"][1][sl] = fy - y0.astype(jnp.float32)
                bk["frac"][2][sl] = fz - z0.astype(jnp.float32)
                b = x0 * sx + y0 * sy + z0
                bk["idx"][0][sl] = b
                bk["idx"][1][sl] = b + sy
                bk["idx"][2][sl] = b + sx
                bk["idx"][3][sl] = b + (sx + sy)
                return c

            lax.fori_loop(0, CHUNK // LANES, index_body, 0)

        def gathers(bk):
            return (
                [pltpu.make_async_copy(p0_h.at[bk["idx"][k]],
                                       bk["res"][k].at[0], bk["sem"])
                 for k in range(4)]
                + [pltpu.make_async_copy(p1_h.at[bk["idx"][k]],
                                         bk["res"][4 + k].at[0], bk["sem"])
                   for k in range(4)]
            )

        def fire(bk):
            for cp in gathers(bk):
                cp.start()

        def drain(bk):
            for cp in gathers(bk):
                cp.wait()

        def combine_store(j, bk):
            base = chunk_base(j)
            res_v = bk["res"]
            lo_v = bk["lo"]
            res_f = [r.bitcast(jnp.float32) for r in res_v]
            lo_f = [r.bitcast(jnp.float32) for r in lo_v]

            def split_body(g, c):
                sl = pl.ds(g * LANES, LANES)
                for k in range(8):
                    w = res_v[k][0, sl]
                    lo_v[k][0, sl] = w << 16
                    res_v[k][0, sl] = w & jnp.int32(-65536)
                return c

            lax.fori_loop(0, CHUNK // LANES, split_body, 0)

            def combine_body(g, c):
                sl = pl.ds(g * LANES, LANES)
                tx = bk["frac"][0][sl]
                ty = bk["frac"][1][sl]
                tz = bk["frac"][2][sl]
                ux = 1.0 - tx
                uy = 1.0 - ty
                uz = 1.0 - tz
                q0 = ux * uy
                q1 = ux * ty
                q2 = tx * uy
                q3 = tx * ty

                def edge(k):
                    return lo_f[k][0, sl] * uz + res_f[k][0, sl] * tz

                m = (q0 * edge(0) + q1 * edge(1)
                     + q2 * edge(2) + q3 * edge(3))
                s = (q0 * edge(4) + q1 * edge(5)
                     + q2 * edge(6) + q3 * edge(7))
                out_v[0][sl] = m
                out_v[1][sl] = s
                return c

            lax.fori_loop(0, CHUNK // LANES, combine_body, 0)
            pltpu.sync_copy(out_v[0], outm_h.at[pl.ds(base, CHUNK)])
            pltpu.sync_copy(out_v[1], outs_h.at[pl.ds(base, CHUNK)])

        load_and_index(0, banks[0])
        fire(banks[0])

        def pair_body(p, carry):
            j0 = 2 * p
            j1 = j0 + 1
            j2 = j0 + 2
            load_and_index(j1, banks[1])
            fire(banks[1])
            drain(banks[0])
            combine_store(j0, banks[0])

            @pl.when(j2 < cpw)
            def _():
                load_and_index(j2, banks[0])
                fire(banks[0])

            drain(banks[1])
            combine_store(j1, banks[1])
            return carry

        lax.fori_loop(0, cpw // 2, pair_body, 0)

    return sc_call


def _pack_pairs_tc(grid, nx, ny, nz):
    rows = 2 * nx * ny
    blk = 4096

    def body(g_ref, o_ref):
        x = g_ref[...]
        lo = lax.bitcast_convert_type(
            x.astype(jnp.bfloat16), jnp.uint16).astype(jnp.uint32)
        hi = pltpu.roll(lo, nz - 1, 1)
        o_ref[...] = lax.bitcast_convert_type(lo | (hi << 16), jnp.int32)

    g2 = grid.reshape(rows, nz)
    out = pl.pallas_call(
        body,
        grid=(rows // blk,),
        in_specs=[pl.BlockSpec((blk, nz), lambda i: (i, 0))],
        out_specs=pl.BlockSpec((blk, nz), lambda i: (i, 0)),
        out_shape=jax.ShapeDtypeStruct((rows, nz), jnp.int32),
    )(g2)
    return out.reshape(-1)


def kernel(pos, grid, min_bound, max_bound):
    n = pos.shape[0]
    _, nx, ny, nz = grid.shape

    n_al = -(-n // 8) * 8
    if n_al != n:
        pos = jnp.concatenate([pos, pos[: n_al - n]])

    posx = pos[:, 0]
    posy = pos[:, 1]
    posz = pos[:, 2]

    grid_range = jnp.clip(max_bound - min_bound, 1e-6, None)
    dims = jnp.array([nx - 1, ny - 1, nz - 1], dtype=jnp.float32)
    scales = dims / grid_range
    params = jnp.concatenate(
        [
            jnp.repeat(min_bound.astype(jnp.float32), LANES),
            jnp.repeat(scales.astype(jnp.float32), LANES),
        ]
    )

    table = _pack_pairs_tc(grid, nx, ny, nz)

    sc_call = _make_sc_call(n_al, nx, ny, nz)
    outm, outs = sc_call(posx, posy, posz, params, table)
    if n_al != n:
        return (outm[:n], outs[:n])
    return (outm, outs)


# speedup vs baseline: 1.4901x; 1.0040x over previous
import functools

import jax
import jax.numpy as jnp
from jax import lax
from jax.experimental import pallas as pl
from jax.experimental.pallas import tpu as pltpu
from jax.experimental.pallas import tpu_sc as plsc

NUM_WORKERS = 32
CHUNK = 2048
LANES = 16
NBUF = 2


def _make_sc_call(n, nx, ny, nz):
    n_chunks = -(-n // CHUNK)
    total_chunks = -(-n_chunks // (2 * NUM_WORKERS)) * (2 * NUM_WORKERS)
    cpw = total_chunks // NUM_WORKERS
    last_base = n - CHUNK
    sx = ny * nz
    sy = nz
    nc = nx * ny * nz

    mesh = plsc.VectorSubcoreMesh(core_axis_name="c", subcore_axis_name="s")

    bank_scratch = (
        [pltpu.VMEM((CHUNK,), jnp.float32) for _ in range(3)]
        + [pltpu.VMEM((CHUNK,), jnp.float32) for _ in range(3)]
        + [pltpu.VMEM((CHUNK,), jnp.int32) for _ in range(4)]
        + [pltpu.VMEM((1, CHUNK), jnp.int32) for _ in range(8)]
        + [pltpu.VMEM((1, CHUNK), jnp.int32) for _ in range(8)]
        + [pltpu.SemaphoreType.DMA]
    )
    scratch = (
        bank_scratch * NBUF
        + [pltpu.VMEM((CHUNK,), jnp.float32) for _ in range(2)]
        + [pltpu.VMEM((LANES,), jnp.float32) for _ in range(6)]
    )

    @functools.partial(
        pl.kernel,
        mesh=mesh,
        out_type=(
            jax.ShapeDtypeStruct((n,), jnp.float32),
            jax.ShapeDtypeStruct((n,), jnp.float32),
        ),
        scratch_types=scratch,
    )
    def sc_call(posx_h, posy_h, posz_h, par_h, tab_h,
                outm_h, outs_h, *refs):
        p0_h = tab_h.at[pl.ds(0, nc)]
        p1_h = tab_h.at[pl.ds(nc, nc)]
        nb = 27
        banks = []
        for b in range(NBUF):
            r = refs[b * nb:(b + 1) * nb]
            banks.append(dict(pos=r[0:3], frac=r[3:6], idx=r[6:10],
                              res=r[10:18], lo=r[18:26], sem=r[26]))
        out_v = refs[2 * nb:2 * nb + 2]
        par_v = refs[2 * nb + 2:2 * nb + 8]

        wid = lax.axis_index("s") * 2 + lax.axis_index("c")

        for d in range(6):
            pltpu.sync_copy(par_h.at[pl.ds(d * LANES, LANES)], par_v[d])
        minx = par_v[0][:]
        miny = par_v[1][:]
        minz = par_v[2][:]
        sclx = par_v[3][:]
        scly = par_v[4][:]
        sclz = par_v[5][:]

        def chunk_base(j):
            t = j * NUM_WORKERS + wid
            return jnp.minimum(t * CHUNK, last_base)

        def load_and_index(j, bk):
            base = chunk_base(j)
            pltpu.sync_copy(posx_h.at[pl.ds(base, CHUNK)], bk["pos"][0])
            pltpu.sync_copy(posy_h.at[pl.ds(base, CHUNK)], bk["pos"][1])
            pltpu.sync_copy(posz_h.at[pl.ds(base, CHUNK)], bk["pos"][2])

            def index_body(g, c):
                sl = pl.ds(g * LANES, LANES)
                fx = jnp.maximum((bk["pos"][0][sl] - minx) * sclx, 0.0)
                fy = jnp.maximum((bk["pos"][1][sl] - miny) * scly, 0.0)
                fz = jnp.maximum((bk["pos"][2][sl] - minz) * sclz, 0.0)
                x0 = jnp.minimum(fx.astype(jnp.int32), nx - 2)
                y0 = jnp.minimum(fy.astype(jnp.int32), ny - 2)
                z0 = jnp.minimum(fz.astype(jnp.int32), nz - 2)
                bk["frac"][0][sl] = fx - x0.astype(jnp.float32)
                bk["frac"][1][sl] = fy - y0.astype(jnp.float32)
                bk["frac"][2][sl] = fz - z0.astype(jnp.float32)
                b = x0 * sx + y0 * sy + z0
                bk["idx"][0][sl] = b
                bk["idx"][1][sl] = b + sy
                bk["idx"][2][sl] = b + sx
                bk["idx"][3][sl] = b + (sx + sy)
                return c

            lax.fori_loop(0, CHUNK // LANES, index_body, 0)

        def gathers(bk):
            return (
                [pltpu.make_async_copy(p0_h.at[bk["idx"][k]],
                                       bk["res"][k].at[0], bk["sem"])
                 for k in range(4)]
                + [pltpu.make_async_copy(p1_h.at[bk["idx"][k]],
                                         bk["res"][4 + k].at[0], bk["sem"])
                   for k in range(4)]
            )

        def fire(bk):
            for cp in gathers(bk):
                cp.start()

        def drain(bk):
            for cp in gathers(bk):
                cp.wait()

        def combine_store(j, bk):
            base = chunk_base(j)
            res_v = bk["res"]
            lo_v = bk["lo"]
            res_f = [r.bitcast(jnp.float32) for r in res_v]
            lo_f = [r.bitcast(jnp.float32) for r in lo_v]

            def split_body(g, c):
                sl = pl.ds(g * LANES, LANES)
                for k in range(8):
                    w = res_v[k][0, sl]
                    lo_v[k][0, sl] = w << 16
                    res_v[k][0, sl] = w & jnp.int32(-65536)
                return c

            lax.fori_loop(0, CHUNK // LANES, split_body, 0)

            def combine_body(g, c):
                sl = pl.ds(g * LANES, LANES)
                tx = bk["frac"][0][sl]
                ty = bk["frac"][1][sl]
                tz = bk["frac"][2][sl]
                ux = 1.0 - tx
                uy = 1.0 - ty
                uz = 1.0 - tz
                q0 = ux * uy
                q1 = ux * ty
                q2 = tx * uy
                q3 = tx * ty

                def edge(k):
                    return lo_f[k][0, sl] * uz + res_f[k][0, sl] * tz

                m = (q0 * edge(0) + q1 * edge(1)
                     + q2 * edge(2) + q3 * edge(3))
                s = (q0 * edge(4) + q1 * edge(5)
                     + q2 * edge(6) + q3 * edge(7))
                out_v[0][sl] = m
                out_v[1][sl] = s
                return c

            lax.fori_loop(0, CHUNK // LANES, combine_body, 0)
            pltpu.sync_copy(out_v[0], outm_h.at[pl.ds(base, CHUNK)])
            pltpu.sync_copy(out_v[1], outs_h.at[pl.ds(base, CHUNK)])

        load_and_index(0, banks[0])
        fire(banks[0])

        def pair_body(p, carry):
            j0 = 2 * p
            j1 = j0 + 1
            j2 = j0 + 2
            load_and_index(j1, banks[1])
            fire(banks[1])
            drain(banks[0])
            combine_store(j0, banks[0])

            @pl.when(j2 < cpw)
            def _():
                load_and_index(j2, banks[0])
                fire(banks[0])

            drain(banks[1])
            combine_store(j1, banks[1])
            return carry

        lax.fori_loop(0, cpw // 2, pair_body, 0)

    return sc_call


def _pack_pairs_tc(grid, nx, ny, nz):
    rows = 2 * nx * ny
    blk = 8192

    def body(g_ref, o_ref):
        x = g_ref[...]
        lo = lax.bitcast_convert_type(
            x.astype(jnp.bfloat16), jnp.uint16).astype(jnp.uint32)
        hi = pltpu.roll(lo, nz - 1, 1)
        o_ref[...] = lax.bitcast_convert_type(lo | (hi << 16), jnp.int32)

    g2 = grid.reshape(rows, nz)
    out = pl.pallas_call(
        body,
        grid=(rows // blk,),
        in_specs=[pl.BlockSpec((blk, nz), lambda i: (i, 0))],
        out_specs=pl.BlockSpec((blk, nz), lambda i: (i, 0)),
        out_shape=jax.ShapeDtypeStruct((rows, nz), jnp.int32),
    )(g2)
    return out.reshape(-1)


def kernel(pos, grid, min_bound, max_bound):
    n = pos.shape[0]
    _, nx, ny, nz = grid.shape

    n_al = -(-n // 8) * 8
    if n_al != n:
        pos = jnp.concatenate([pos, pos[: n_al - n]])

    posx = pos[:, 0]
    posy = pos[:, 1]
    posz = pos[:, 2]

    grid_range = jnp.clip(max_bound - min_bound, 1e-6, None)
    dims = jnp.array([nx - 1, ny - 1, nz - 1], dtype=jnp.float32)
    scales = dims / grid_range
    params = jnp.concatenate(
        [
            jnp.repeat(min_bound.astype(jnp.float32), LANES),
            jnp.repeat(scales.astype(jnp.float32), LANES),
        ]
    )

    table = _pack_pairs_tc(grid, nx, ny, nz)

    sc_call = _make_sc_call(n_al, nx, ny, nz)
    outm, outs = sc_call(posx, posy, posz, params, table)
    if n_al != n:
        return (outm[:n], outs[:n])
    return (outm, outs)
